# Initial kernel scaffold; baseline (speedup 1.0000x reference)
import jax
import jax.numpy as jnp
from jax.experimental import pallas as pl


def kernel(phrase_embed, box_features, target_ids, main_w1pd, main_w1p, main_w1b, main_b1, main_w2t, main_b2, topn_w1pd, topn_w1p, topn_w1b, topn_b1, topn_w2t, topn_b2):
    raise NotImplementedError("write your pallas kernel here")


# speedup vs baseline: 1.2466x; 1.2466x over previous
import jax
import jax.numpy as jnp
from jax.experimental import pallas as pl
from jax.experimental.pallas import tpu as pltpu

_TOPN = 16


def _mlp_pairs_body(phr, box, w1pd_ref, w1p_ref, w1b_ref, b1_ref,
                    w2t_ref, b2_ref):
    P, D = phr.shape
    NBOX = box.shape[1]
    H2 = b1_ref.shape[1]
    R = P * NBOX
    bf16 = jnp.bfloat16

    hp = jnp.dot(phr.astype(bf16), w1p_ref[...],
                 preferred_element_type=jnp.float32)
    hb = jnp.dot(box.reshape(-1, D).astype(bf16), w1b_ref[...],
                 preferred_element_type=jnp.float32)

    prod = (phr[:, None, :] * box).astype(bf16)
    diff = (phr[:, None, :] - box).astype(bf16)
    pd = jnp.concatenate([prod, diff], axis=-1).reshape(R, 2 * D)

    h = jnp.dot(pd, w1pd_ref[...], preferred_element_type=jnp.float32)
    if box.shape[0] == 1:
        h = h.reshape(P, NBOX, H2) + hp[:, None, :] + hb[None] + b1_ref[...]
    else:
        h = h + hb + b1_ref[...]
        h = h.reshape(P, NBOX, H2) + hp[:, None, :]
    h = jnp.maximum(h, 0.01 * h)

    out = jnp.einsum('ok,rk->or', w2t_ref[...], h.reshape(R, H2),
                     preferred_element_type=jnp.float32) + b2_ref[...]
    return out


def _allpairs_kernel(phr_ref, box_ref, w1pd_ref, w1p_ref, w1b_ref,
                     b1_ref, w2t_ref, b2_ref, out_ref):
    out = _mlp_pairs_body(phr_ref[0], box_ref[...], w1pd_ref, w1p_ref,
                          w1b_ref, b1_ref, w2t_ref, b2_ref)
    out_ref[...] = out[None]


def _topn_kernel(phr_ref, box_ref, w1pd_ref, w1p_ref, w1b_ref,
                 b1_ref, w2t_ref, b2_ref, out_ref):
    out_ref[...] = _mlp_pairs_body(phr_ref[...], box_ref[...], w1pd_ref,
                                   w1p_ref, w1b_ref, b1_ref, w2t_ref, b2_ref)


def _weight_specs(D, H2):
    zmap2 = lambda *a: (0, 0)
    return [
        pl.BlockSpec((2 * D, H2), zmap2),
        pl.BlockSpec((D, H2), zmap2),
        pl.BlockSpec((D, H2), zmap2),
        pl.BlockSpec((1, H2), zmap2),
        pl.BlockSpec((8, H2), zmap2),
        pl.BlockSpec((8, 1), zmap2),
    ]


def kernel(phrase_embed, box_features, target_ids,
           main_w1pd, main_w1p, main_w1b, main_b1, main_w2t, main_b2,
           topn_w1pd, topn_w1p, topn_w1b, topn_b1, topn_w2t, topn_b2):
    B, P, D = phrase_embed.shape
    NB = box_features.shape[1]
    H2 = main_b1.shape[1]
    T = _TOPN
    R = P * NB

    flops = (2 * B * R * (2 * D * H2 + H2 * 8)
             + 2 * B * (P + NB) * D * H2)
    cost = pl.CostEstimate(
        flops=flops, transcendentals=0,
        bytes_accessed=(B * (P + NB) * D + B * 8 * R) * 4)

    out = pl.pallas_call(
        _allpairs_kernel,
        grid=(B,),
        in_specs=[
            pl.BlockSpec((1, P, D), lambda b: (b, 0, 0)),
            pl.BlockSpec((1, NB, D), lambda b: (b, 0, 0)),
            *_weight_specs(D, H2),
        ],
        out_specs=pl.BlockSpec((1, 8, R), lambda b: (b, 0, 0)),
        out_shape=jax.ShapeDtypeStruct((B, 8, R), jnp.float32),
        compiler_params=pltpu.CompilerParams(
            dimension_semantics=("parallel",)),
        cost_estimate=cost,
    )(phrase_embed, box_features,
      main_w1pd, main_w1p, main_w1b, main_b1, main_w2t, main_b2)

    out = out.reshape(B, 8, P, NB)
    pred_sim = out[:, 0]
    reg_offset = jnp.transpose(out[:, 1:7], (0, 2, 3, 1))

    topN_scores, topN_ids = jax.lax.top_k(pred_sim, T)
    del topN_scores
    feats_topN = jax.vmap(lambda f, idx: f[idx])(box_features, topN_ids)

    BP = B * P
    GR = 2
    BPG = BP // GR
    phr_flat = phrase_embed.reshape(BP, D)
    box_flat = feats_topN.reshape(BP, T, D)

    cost_t = pl.CostEstimate(
        flops=2 * BP * T * (3 * D * H2 + H2 * 8), transcendentals=0,
        bytes_accessed=(BP * D + BP * T * D + 8 * BP * T) * 4)

    out_t = pl.pallas_call(
        _topn_kernel,
        grid=(GR,),
        in_specs=[
            pl.BlockSpec((BPG, D), lambda i: (i, 0)),
            pl.BlockSpec((BPG, T, D), lambda i: (i, 0, 0)),
            *_weight_specs(D, H2),
        ],
        out_specs=pl.BlockSpec((8, BPG * T), lambda i: (0, i)),
        out_shape=jax.ShapeDtypeStruct((8, BP * T), jnp.float32),
        compiler_params=pltpu.CompilerParams(
            dimension_semantics=("parallel",)),
        cost_estimate=cost_t,
    )(phr_flat, box_flat,
      topn_w1pd, topn_w1p, topn_w1b, topn_b1, topn_w2t, topn_b2)

    out_t = out_t.reshape(8, B, P, T)
    pred_sim_topN = out_t[0]
    reg_offset_topN = jnp.transpose(out_t[1:7], (1, 2, 3, 0))

    bidx = jnp.arange(B)
    res = {
        "batch_pred_similarity":      pred_sim[bidx, target_ids],
        "batch_reg_offset":           reg_offset[bidx, target_ids],
        "batch_pred_similarity_topN": pred_sim_topN[bidx, target_ids],
        "batch_reg_offset_topN":      reg_offset_topN[bidx, target_ids],
        "batch_topN_target_ids":      topN_ids[bidx, target_ids],
        "batch_target_embed":         phrase_embed[bidx, target_ids],
    }
    return res, (pred_sim, reg_offset, pred_sim_topN, reg_offset_topN, topN_ids)


# speedup vs baseline: 1.3326x; 1.0690x over previous
import jax
import jax.numpy as jnp
from jax.experimental import pallas as pl
from jax.experimental.pallas import tpu as pltpu

_TOPN = 16
_CORES = 2


def _pipe_kernel(phr_ref, box_ref, phrp_ref, boxp_ref,
                 mw1pd_ref, mw1p_ref, mw1b_ref, mb1_ref, mw2t_ref, mb2_ref,
                 tw1pd_ref, tw1p_ref, tw1b_ref, tb1_ref, tw2t_ref, tb2_ref,
                 out_ref, ids_ref, outt_ref, sim_scr):
    P, D = phr_ref.shape[1], phr_ref.shape[2]
    NB = box_ref.shape[1]
    H2 = mb1_ref.shape[1]
    T = _TOPN
    R = P * NB
    bf16 = jnp.bfloat16

    if True:
        sim = sim_scr[...]
        iota = jax.lax.broadcasted_iota(jnp.int32, (P, NB), 1)
        big = jnp.int32(1 << 30)
        ninf = jnp.float32(-jnp.inf)
        x = sim
        idx_cols = []
        for _ in range(T):
            m = jnp.max(x, axis=1, keepdims=True)
            idx = jnp.min(jnp.where(x == m, iota, big), axis=1, keepdims=True)
            idx_cols.append(idx)
            x = jnp.where(iota == idx, ninf, x)
        ids = jnp.concatenate(idx_cols, axis=1)
        ids_ref[...] = ids[None]

        boxp = boxp_ref[0]
        phrp = phrp_ref[0]
        idxcol = jnp.concatenate(idx_cols, axis=0)
        row_iota = jax.lax.broadcasted_iota(jnp.int32, (T * P, NB), 1)
        onehot = (row_iota == idxcol).astype(bf16)
        feats = jnp.dot(onehot, boxp.astype(bf16),
                        preferred_element_type=jnp.float32)

        phrrep = jnp.concatenate([phrp] * T, axis=0)
        hpt = jnp.dot(phrp.astype(bf16), tw1p_ref[...],
                      preferred_element_type=jnp.float32)
        hpt_rep = jnp.concatenate([hpt] * T, axis=0)
        hbt = jnp.dot(feats.astype(bf16), tw1b_ref[...],
                      preferred_element_type=jnp.float32)

        prodt = (phrrep * feats).astype(bf16)
        difft = (phrrep - feats).astype(bf16)
        pdt = jnp.concatenate([prodt, difft], axis=-1)

        ht = jnp.dot(pdt, tw1pd_ref[...], preferred_element_type=jnp.float32)
        ht = ht + hbt + tb1_ref[...]
        ht = ht + hpt_rep
        ht = jnp.maximum(ht, 0.01 * ht)

        outt = jnp.einsum('ok,rk->or', tw2t_ref[...], ht,
                          preferred_element_type=jnp.float32) + tb2_ref[...]
        outt_ref[...] = outt[None]

    if True:
        phr = phr_ref[0]
        box = box_ref[0]

        hp = jnp.dot(phr.astype(bf16), mw1p_ref[...],
                     preferred_element_type=jnp.float32)
        hb = jnp.dot(box.astype(bf16), mw1b_ref[...],
                     preferred_element_type=jnp.float32)

        prod = (phr[:, None, :] * box[None, :, :]).astype(bf16)
        diff = (phr[:, None, :] - box[None, :, :]).astype(bf16)
        pd = jnp.concatenate([prod, diff], axis=-1).reshape(R, 2 * D)

        h = jnp.dot(pd, mw1pd_ref[...], preferred_element_type=jnp.float32)
        h = (h.reshape(P, NB, H2) + hp[:, None, :] + hb[None, :, :]
             + mb1_ref[...])
        h = jnp.maximum(h, 0.01 * h)

        out = jnp.einsum('ok,rk->or', mw2t_ref[...], h.reshape(R, H2),
                         preferred_element_type=jnp.float32) + mb2_ref[...]
        out_ref[...] = out[None]

        simrow = out[0:1, :]
        sim_scr[...] = jnp.concatenate(
            [simrow[:, p * NB:(p + 1) * NB] for p in range(P)], axis=0)


def _weight_specs(D, H2):
    zmap2 = lambda *a: (0, 0)
    return [
        pl.BlockSpec((2 * D, H2), zmap2),
        pl.BlockSpec((D, H2), zmap2),
        pl.BlockSpec((D, H2), zmap2),
        pl.BlockSpec((1, H2), zmap2),
        pl.BlockSpec((8, H2), zmap2),
        pl.BlockSpec((8, 1), zmap2),
    ]


def kernel(phrase_embed, box_features, target_ids,
           main_w1pd, main_w1p, main_w1b, main_b1, main_w2t, main_b2,
           topn_w1pd, topn_w1p, topn_w1b, topn_b1, topn_w2t, topn_b2):
    B, P, D = phrase_embed.shape
    NB = box_features.shape[1]
    H2 = main_b1.shape[1]
    T = _TOPN
    R = P * NB
    CB = B // _CORES

    def cur3(c, j):
        return (jnp.minimum(c * CB + j, c * CB + CB - 1), 0, 0)

    def prev3(c, j):
        return (jnp.maximum(c * CB + j - 1, c * CB), 0, 0)

    def wmap(*a):
        return (0, 0)

    flops = (2 * B * R * (2 * D * H2 + H2 * 8)
             + 2 * B * (P + NB) * D * H2
             + 2 * B * T * P * (NB * D + 3 * D * H2 + H2 * 8))
    cost = pl.CostEstimate(
        flops=flops, transcendentals=0,
        bytes_accessed=(2 * B * (P + NB) * D + B * 8 * R) * 4)

    out, ids, outt = pl.pallas_call(
        _pipe_kernel,
        grid=(_CORES, CB + 1),
        in_specs=[
            pl.BlockSpec((1, P, D), cur3),
            pl.BlockSpec((1, NB, D), cur3),
            pl.BlockSpec((1, P, D), prev3),
            pl.BlockSpec((1, NB, D), prev3),
            *_weight_specs(D, H2),
            *_weight_specs(D, H2),
        ],
        out_specs=[
            pl.BlockSpec((1, 8, R), cur3),
            pl.BlockSpec((1, P, T), prev3),
            pl.BlockSpec((1, 8, T * P), prev3),
        ],
        out_shape=[
            jax.ShapeDtypeStruct((B, 8, R), jnp.float32),
            jax.ShapeDtypeStruct((B, P, T), jnp.int32),
            jax.ShapeDtypeStruct((B, 8, T * P), jnp.float32),
        ],
        scratch_shapes=[pltpu.VMEM((P, NB), jnp.float32)],
        compiler_params=pltpu.CompilerParams(
            dimension_semantics=("parallel", "arbitrary")),
        cost_estimate=cost,
    )(phrase_embed, box_features, phrase_embed, box_features,
      main_w1pd, main_w1p, main_w1b, main_b1, main_w2t, main_b2,
      topn_w1pd, topn_w1p, topn_w1b, topn_b1, topn_w2t, topn_b2)

    out = out.reshape(B, 8, P, NB)
    pred_sim = out[:, 0]
    reg_offset = jnp.transpose(out[:, 1:7], (0, 2, 3, 1))

    outt = jnp.transpose(outt.reshape(B, 8, T, P), (0, 1, 3, 2))
    pred_sim_topN = outt[:, 0]
    reg_offset_topN = jnp.transpose(outt[:, 1:7], (0, 2, 3, 1))

    bidx = jnp.arange(B)
    res = {
        "batch_pred_similarity":      pred_sim[bidx, target_ids],
        "batch_reg_offset":           reg_offset[bidx, target_ids],
        "batch_pred_similarity_topN": pred_sim_topN[bidx, target_ids],
        "batch_reg_offset_topN":      reg_offset_topN[bidx, target_ids],
        "batch_topN_target_ids":      ids[bidx, target_ids],
        "batch_target_embed":         phrase_embed[bidx, target_ids],
    }
    return res, (pred_sim, reg_offset, pred_sim_topN, reg_offset_topN, ids)


# speedup vs baseline: 1.6581x; 1.2443x over previous
import jax
import jax.numpy as jnp
from jax.experimental import pallas as pl
from jax.experimental.pallas import tpu as pltpu

_TOPN = 16
_CORES = 2


def _pipe_kernel(phr_ref, box_ref, phrp_ref, boxp_ref,
                 mw1pd_ref, mw1p_ref, mw1b_ref, mb1_ref, mw2t_ref, mb2_ref,
                 tw1pd_ref, tw1p_ref, tw1b_ref, tb1_ref, tw2t_ref, tb2_ref,
                 out_ref, ids_ref, outt_ref, sim_scr):
    P, D = phr_ref.shape[1], phr_ref.shape[2]
    NB = box_ref.shape[1]
    H2 = mb1_ref.shape[1]
    T = _TOPN
    R = P * NB
    bf16 = jnp.bfloat16

    phr = phr_ref[0]
    box = box_ref[0]

    hp = jnp.dot(phr.astype(bf16), mw1p_ref[...],
                 preferred_element_type=jnp.float32)
    hb = jnp.dot(box.astype(bf16), mw1b_ref[...],
                 preferred_element_type=jnp.float32)

    prod = (phr[:, None, :] * box[None, :, :]).astype(bf16)
    diff = (phr[:, None, :] - box[None, :, :]).astype(bf16)
    pd = jnp.concatenate([prod, diff], axis=-1).reshape(R, 2 * D)

    h = jnp.dot(pd, mw1pd_ref[...], preferred_element_type=jnp.float32)

    sim = sim_scr[...]
    iota = jax.lax.broadcasted_iota(jnp.int32, (P, NB), 1)
    big = jnp.int32(1 << 30)
    ninf = jnp.float32(-jnp.inf)
    x = sim
    idx_cols = []
    for _ in range(T):
        m = jnp.max(x, axis=1, keepdims=True)
        idx = jnp.min(jnp.where(x == m, iota, big), axis=1, keepdims=True)
        idx_cols.append(idx)
        x = jnp.where(iota == idx, ninf, x)
    ids = jnp.concatenate(idx_cols, axis=1)
    ids_ref[...] = ids[None]

    boxp = boxp_ref[0]
    phrp = phrp_ref[0]
    idxcol = jnp.concatenate(idx_cols, axis=0)
    row_iota = jax.lax.broadcasted_iota(jnp.int32, (T * P, NB), 1)
    onehot = (row_iota == idxcol).astype(bf16)
    feats = jnp.dot(onehot, boxp.astype(bf16),
                    preferred_element_type=jnp.float32)

    phrrep = jnp.concatenate([phrp] * T, axis=0)
    hpt = jnp.dot(phrp.astype(bf16), tw1p_ref[...],
                  preferred_element_type=jnp.float32)
    hpt_rep = jnp.concatenate([hpt] * T, axis=0)
    hbt = jnp.dot(feats.astype(bf16), tw1b_ref[...],
                  preferred_element_type=jnp.float32)

    prodt = (phrrep * feats).astype(bf16)
    difft = (phrrep - feats).astype(bf16)
    pdt = jnp.concatenate([prodt, difft], axis=-1)

    ht = jnp.dot(pdt, tw1pd_ref[...], preferred_element_type=jnp.float32)
    ht = ht + hbt + tb1_ref[...]
    ht = ht + hpt_rep
    ht = jnp.maximum(ht, 0.01 * ht)

    outt = jnp.einsum('ok,rk->or', tw2t_ref[...], ht,
                      preferred_element_type=jnp.float32) + tb2_ref[...]
    outt_ref[...] = outt[None]

    h = (h.reshape(P, NB, H2) + hp[:, None, :] + hb[None, :, :]
         + mb1_ref[...])
    h = jnp.maximum(h, 0.01 * h)

    out = jnp.einsum('ok,rk->or', mw2t_ref[...], h.reshape(R, H2),
                     preferred_element_type=jnp.float32) + mb2_ref[...]
    out_ref[...] = out[None]

    simrow = out[0:1, :]
    sim_scr[...] = jnp.concatenate(
        [simrow[:, p * NB:(p + 1) * NB] for p in range(P)], axis=0)


def _weight_specs(D, H2):
    zmap2 = lambda *a: (0, 0)
    return [
        pl.BlockSpec((2 * D, H2), zmap2),
        pl.BlockSpec((D, H2), zmap2),
        pl.BlockSpec((D, H2), zmap2),
        pl.BlockSpec((1, H2), zmap2),
        pl.BlockSpec((8, H2), zmap2),
        pl.BlockSpec((8, 1), zmap2),
    ]


def kernel(phrase_embed, box_features, target_ids,
           main_w1pd, main_w1p, main_w1b, main_b1, main_w2t, main_b2,
           topn_w1pd, topn_w1p, topn_w1b, topn_b1, topn_w2t, topn_b2):
    B, P, D = phrase_embed.shape
    NB = box_features.shape[1]
    H2 = main_b1.shape[1]
    T = _TOPN
    R = P * NB
    CB = B // _CORES

    def cur3(c, j):
        return (jnp.minimum(c * CB + j, c * CB + CB - 1), 0, 0)

    def prev3(c, j):
        return (jnp.maximum(c * CB + j - 1, c * CB), 0, 0)

    flops = (2 * B * R * (2 * D * H2 + H2 * 8)
             + 2 * B * (P + NB) * D * H2
             + 2 * B * T * P * (NB * D + 3 * D * H2 + H2 * 8))
    cost = pl.CostEstimate(
        flops=flops, transcendentals=0,
        bytes_accessed=(2 * B * (P + NB) * D + B * 8 * R) * 4)

    out, ids, outt = pl.pallas_call(
        _pipe_kernel,
        grid=(_CORES, CB + 1),
        in_specs=[
            pl.BlockSpec((1, P, D), cur3),
            pl.BlockSpec((1, NB, D), cur3),
            pl.BlockSpec((1, P, D), prev3),
            pl.BlockSpec((1, NB, D), prev3),
            *_weight_specs(D, H2),
            *_weight_specs(D, H2),
        ],
        out_specs=[
            pl.BlockSpec((1, 8, R), cur3),
            pl.BlockSpec((1, P, T), prev3),
            pl.BlockSpec((1, 8, T * P), prev3),
        ],
        out_shape=[
            jax.ShapeDtypeStruct((B, 8, R), jnp.float32),
            jax.ShapeDtypeStruct((B, P, T), jnp.int32),
            jax.ShapeDtypeStruct((B, 8, T * P), jnp.float32),
        ],
        scratch_shapes=[pltpu.VMEM((P, NB), jnp.float32)],
        compiler_params=pltpu.CompilerParams(
            dimension_semantics=("parallel", "arbitrary")),
        cost_estimate=cost,
    )(phrase_embed, box_features, phrase_embed, box_features,
      main_w1pd, main_w1p, main_w1b, main_b1, main_w2t, main_b2,
      topn_w1pd, topn_w1p, topn_w1b, topn_b1, topn_w2t, topn_b2)

    out = out.reshape(B, 8, P, NB)
    pred_sim = out[:, 0]
    reg_offset = jnp.transpose(out[:, 1:7], (0, 2, 3, 1))

    outt = jnp.transpose(outt.reshape(B, 8, T, P), (0, 1, 3, 2))
    pred_sim_topN = outt[:, 0]
    reg_offset_topN = jnp.transpose(outt[:, 1:7], (0, 2, 3, 1))

    bidx = jnp.arange(B)
    res = {
        "batch_pred_similarity":      pred_sim[bidx, target_ids],
        "batch_reg_offset":           reg_offset[bidx, target_ids],
        "batch_pred_similarity_topN": pred_sim_topN[bidx, target_ids],
        "batch_reg_offset_topN":      reg_offset_topN[bidx, target_ids],
        "batch_topN_target_ids":      ids[bidx, target_ids],
        "batch_target_embed":         phrase_embed[bidx, target_ids],
    }
    return res, (pred_sim, reg_offset, pred_sim_topN, reg_offset_topN, ids)


# speedup vs baseline: 1.6822x; 1.0145x over previous
import jax
import jax.numpy as jnp
from jax.experimental import pallas as pl
from jax.experimental.pallas import tpu as pltpu

_TOPN = 16
_CORES = 2


def _pipe_kernel(phr_ref, box_ref, phrp_ref, boxp_ref,
                 mw1pd_ref, mw1p_ref, mw1b_ref, mb1_ref, mw2t_ref, mb2_ref,
                 tw1pd_ref, tw1p_ref, tw1b_ref, tb1_ref, tw2t_ref, tb2_ref,
                 out_ref, ids_ref, outt_ref, sim_out_ref, sim_scr):
    P, D = phr_ref.shape[1], phr_ref.shape[2]
    NB = box_ref.shape[1]
    H2 = mb1_ref.shape[1]
    T = _TOPN
    R = P * NB
    bf16 = jnp.bfloat16

    phr = phr_ref[0]
    box = box_ref[0]

    hp = jnp.dot(phr.astype(bf16), mw1p_ref[...],
                 preferred_element_type=jnp.float32)
    hb = jnp.dot(box.astype(bf16), mw1b_ref[...],
                 preferred_element_type=jnp.float32)

    prod = (phr[:, None, :] * box[None, :, :]).astype(bf16)
    diff = (phr[:, None, :] - box[None, :, :]).astype(bf16)
    pd = jnp.concatenate([prod, diff], axis=-1).reshape(R, 2 * D)

    h = jnp.dot(pd, mw1pd_ref[...], preferred_element_type=jnp.float32)

    sim = sim_scr[...]
    iota = jax.lax.broadcasted_iota(jnp.int32, (P, NB), 1)
    big = jnp.int32(1 << 30)
    ninf = jnp.float32(-jnp.inf)
    x = sim
    idx_cols = []
    for _ in range(T):
        m = jnp.max(x, axis=1, keepdims=True)
        idx = jnp.min(jnp.where(x == m, iota, big), axis=1, keepdims=True)
        idx_cols.append(idx)
        x = jnp.where(iota == idx, ninf, x)
    ids = jnp.concatenate(idx_cols, axis=1)
    ids_ref[...] = ids[None]

    boxp = boxp_ref[0]
    phrp = phrp_ref[0]
    idxcol = jnp.concatenate(idx_cols, axis=0)
    row_iota = jax.lax.broadcasted_iota(jnp.int32, (T * P, NB), 1)
    onehot = (row_iota == idxcol).astype(bf16)
    feats = jnp.dot(onehot, boxp.astype(bf16),
                    preferred_element_type=jnp.float32)

    phrrep = jnp.concatenate([phrp] * T, axis=0)
    hpt = jnp.dot(phrp.astype(bf16), tw1p_ref[...],
                  preferred_element_type=jnp.float32)
    hpt_rep = jnp.concatenate([hpt] * T, axis=0)
    hbt = jnp.dot(feats.astype(bf16), tw1b_ref[...],
                  preferred_element_type=jnp.float32)

    prodt = (phrrep * feats).astype(bf16)
    difft = (phrrep - feats).astype(bf16)
    pdt = jnp.concatenate([prodt, difft], axis=-1)

    ht = jnp.dot(pdt, tw1pd_ref[...], preferred_element_type=jnp.float32)
    ht = ht + hbt + tb1_ref[...]
    ht = ht + hpt_rep
    ht = jnp.maximum(ht, 0.01 * ht)

    outt = jnp.einsum('ok,rk->or', tw2t_ref[...], ht,
                      preferred_element_type=jnp.float32) + tb2_ref[...]
    outt_ref[...] = outt[None]

    h = (h.reshape(P, NB, H2) + hp[:, None, :] + hb[None, :, :]
         + mb1_ref[...])
    h = jnp.maximum(h, 0.01 * h)

    out = jnp.einsum('ok,rk->or', mw2t_ref[...], h.reshape(R, H2),
                     preferred_element_type=jnp.float32) + mb2_ref[...]
    out_ref[...] = out[None]

    simrow = out[0:1, :]
    simc = jnp.concatenate(
        [simrow[:, p * NB:(p + 1) * NB] for p in range(P)], axis=0)
    sim_out_ref[...] = simc[None]
    sim_scr[...] = simc


def _weight_specs(D, H2):
    zmap2 = lambda *a: (0, 0)
    return [
        pl.BlockSpec((2 * D, H2), zmap2),
        pl.BlockSpec((D, H2), zmap2),
        pl.BlockSpec((D, H2), zmap2),
        pl.BlockSpec((1, H2), zmap2),
        pl.BlockSpec((8, H2), zmap2),
        pl.BlockSpec((8, 1), zmap2),
    ]


def kernel(phrase_embed, box_features, target_ids,
           main_w1pd, main_w1p, main_w1b, main_b1, main_w2t, main_b2,
           topn_w1pd, topn_w1p, topn_w1b, topn_b1, topn_w2t, topn_b2):
    B, P, D = phrase_embed.shape
    NB = box_features.shape[1]
    H2 = main_b1.shape[1]
    T = _TOPN
    R = P * NB
    CB = B // _CORES

    def cur3(c, j):
        return (jnp.minimum(c * CB + j, c * CB + CB - 1), 0, 0)

    def prev3(c, j):
        return (jnp.maximum(c * CB + j - 1, c * CB), 0, 0)

    flops = (2 * B * R * (2 * D * H2 + H2 * 8)
             + 2 * B * (P + NB) * D * H2
             + 2 * B * T * P * (NB * D + 3 * D * H2 + H2 * 8))
    cost = pl.CostEstimate(
        flops=flops, transcendentals=0,
        bytes_accessed=(2 * B * (P + NB) * D + B * 8 * R) * 4)

    out, ids, outt, pred_sim = pl.pallas_call(
        _pipe_kernel,
        grid=(_CORES, CB + 1),
        in_specs=[
            pl.BlockSpec((1, P, D), cur3),
            pl.BlockSpec((1, NB, D), cur3),
            pl.BlockSpec((1, P, D), prev3),
            pl.BlockSpec((1, NB, D), prev3),
            *_weight_specs(D, H2),
            *_weight_specs(D, H2),
        ],
        out_specs=[
            pl.BlockSpec((1, 8, R), cur3),
            pl.BlockSpec((1, P, T), prev3),
            pl.BlockSpec((1, 8, T * P), prev3),
            pl.BlockSpec((1, P, NB), cur3),
        ],
        out_shape=[
            jax.ShapeDtypeStruct((B, 8, R), jnp.float32),
            jax.ShapeDtypeStruct((B, P, T), jnp.int32),
            jax.ShapeDtypeStruct((B, 8, T * P), jnp.float32),
            jax.ShapeDtypeStruct((B, P, NB), jnp.float32),
        ],
        scratch_shapes=[pltpu.VMEM((P, NB), jnp.float32)],
        compiler_params=pltpu.CompilerParams(
            dimension_semantics=("parallel", "arbitrary")),
        cost_estimate=cost,
    )(phrase_embed, box_features, phrase_embed, box_features,
      main_w1pd, main_w1p, main_w1b, main_b1, main_w2t, main_b2,
      topn_w1pd, topn_w1p, topn_w1b, topn_b1, topn_w2t, topn_b2)

    out = out.reshape(B, 8, P, NB)
    reg_offset = jnp.transpose(out[:, 1:7], (0, 2, 3, 1))

    outt = jnp.transpose(outt.reshape(B, 8, T, P), (0, 1, 3, 2))
    pred_sim_topN = outt[:, 0]
    reg_offset_topN = jnp.transpose(outt[:, 1:7], (0, 2, 3, 1))

    bidx = jnp.arange(B)
    res = {
        "batch_pred_similarity":      pred_sim[bidx, target_ids],
        "batch_reg_offset":           reg_offset[bidx, target_ids],
        "batch_pred_similarity_topN": pred_sim_topN[bidx, target_ids],
        "batch_reg_offset_topN":      reg_offset_topN[bidx, target_ids],
        "batch_topN_target_ids":      ids[bidx, target_ids],
        "batch_target_embed":         phrase_embed[bidx, target_ids],
    }
    return res, (pred_sim, reg_offset, pred_sim_topN, reg_offset_topN, ids)


# speedup vs baseline: 1.9578x; 1.1638x over previous
import jax
import jax.numpy as jnp
from jax.experimental import pallas as pl
from jax.experimental.pallas import tpu as pltpu

_TOPN = 16
_CORES = 2


def _pipe_kernel(phr_ref, box_ref, phrp_ref, boxp_ref,
                 mw1pd_ref, mw1p_ref, mw1b_ref, mb1_ref, mw2t_ref, mb2_ref,
                 tw1pd_ref, tw1p_ref, tw1b_ref, tb1_ref, tw2t_ref, tb2_ref,
                 out_ref, ids_ref, outt_ref, sim_out_ref, sim_scr):
    P, D = phr_ref.shape[1], phr_ref.shape[2]
    NB = box_ref.shape[1]
    H2 = mb1_ref.shape[1]
    T = _TOPN
    R = P * NB
    bf16 = jnp.bfloat16

    phr = phr_ref[0]
    box = box_ref[0]

    hp = jnp.dot(phr.astype(bf16), mw1p_ref[...],
                 preferred_element_type=jnp.float32)
    hb = jnp.dot(box.astype(bf16), mw1b_ref[...],
                 preferred_element_type=jnp.float32)

    prod = (phr[:, None, :] * box[None, :, :]).astype(bf16)
    diff = (phr[:, None, :] - box[None, :, :]).astype(bf16)
    pd = jnp.concatenate([prod, diff], axis=-1).reshape(R, 2 * D)

    h = jnp.dot(pd, mw1pd_ref[...], preferred_element_type=jnp.float32)

    sim = sim_scr[...]
    iota = jax.lax.broadcasted_iota(jnp.int32, (P, NB), 1)
    ninf = jnp.float32(-jnp.inf)
    x = sim
    idx_cols = []
    for _ in range(T):
        idx = jnp.argmax(x, axis=1).astype(jnp.int32)[:, None]
        idx_cols.append(idx)
        x = jnp.where(iota == idx, ninf, x)
    ids = jnp.concatenate(idx_cols, axis=1)
    ids_ref[...] = ids[None]

    boxp = boxp_ref[0]
    phrp = phrp_ref[0]
    idxcol = jnp.concatenate(idx_cols, axis=0)
    row_iota = jax.lax.broadcasted_iota(jnp.int32, (T * P, NB), 1)
    onehot = (row_iota == idxcol).astype(bf16)
    feats = jnp.dot(onehot, boxp.astype(bf16),
                    preferred_element_type=jnp.float32)

    phrrep = jnp.concatenate([phrp] * T, axis=0)
    hpt = jnp.dot(phrp.astype(bf16), tw1p_ref[...],
                  preferred_element_type=jnp.float32)
    hpt_rep = jnp.concatenate([hpt] * T, axis=0)
    hbt = jnp.dot(feats.astype(bf16), tw1b_ref[...],
                  preferred_element_type=jnp.float32)

    prodt = (phrrep * feats).astype(bf16)
    difft = (phrrep - feats).astype(bf16)
    pdt = jnp.concatenate([prodt, difft], axis=-1)

    ht = jnp.dot(pdt, tw1pd_ref[...], preferred_element_type=jnp.float32)
    ht = ht + hbt + tb1_ref[...]
    ht = ht + hpt_rep
    ht = jnp.maximum(ht, 0.01 * ht)

    outt = jnp.einsum('ok,rk->or', tw2t_ref[...], ht,
                      preferred_element_type=jnp.float32) + tb2_ref[...]
    outt_ref[...] = outt[None]

    h = (h.reshape(P, NB, H2) + hp[:, None, :] + hb[None, :, :]
         + mb1_ref[...])
    h = jnp.maximum(h, 0.01 * h)

    out = jnp.einsum('ok,rk->or', mw2t_ref[...], h.reshape(R, H2),
                     preferred_element_type=jnp.float32) + mb2_ref[...]
    out_ref[...] = out[None]

    simrow = out[0:1, :]
    simc = jnp.concatenate(
        [simrow[:, p * NB:(p + 1) * NB] for p in range(P)], axis=0)
    sim_out_ref[...] = simc[None]
    sim_scr[...] = simc


def _weight_specs(D, H2):
    zmap2 = lambda *a: (0, 0)
    return [
        pl.BlockSpec((2 * D, H2), zmap2),
        pl.BlockSpec((D, H2), zmap2),
        pl.BlockSpec((D, H2), zmap2),
        pl.BlockSpec((1, H2), zmap2),
        pl.BlockSpec((8, H2), zmap2),
        pl.BlockSpec((8, 1), zmap2),
    ]


def kernel(phrase_embed, box_features, target_ids,
           main_w1pd, main_w1p, main_w1b, main_b1, main_w2t, main_b2,
           topn_w1pd, topn_w1p, topn_w1b, topn_b1, topn_w2t, topn_b2):
    B, P, D = phrase_embed.shape
    NB = box_features.shape[1]
    H2 = main_b1.shape[1]
    T = _TOPN
    R = P * NB
    CB = B // _CORES

    def cur3(c, j):
        return (jnp.minimum(c * CB + j, c * CB + CB - 1), 0, 0)

    def prev3(c, j):
        return (jnp.maximum(c * CB + j - 1, c * CB), 0, 0)

    flops = (2 * B * R * (2 * D * H2 + H2 * 8)
             + 2 * B * (P + NB) * D * H2
             + 2 * B * T * P * (NB * D + 3 * D * H2 + H2 * 8))
    cost = pl.CostEstimate(
        flops=flops, transcendentals=0,
        bytes_accessed=(2 * B * (P + NB) * D + B * 8 * R) * 4)

    out, ids, outt, pred_sim = pl.pallas_call(
        _pipe_kernel,
        grid=(_CORES, CB + 1),
        in_specs=[
            pl.BlockSpec((1, P, D), cur3),
            pl.BlockSpec((1, NB, D), cur3),
            pl.BlockSpec((1, P, D), prev3),
            pl.BlockSpec((1, NB, D), prev3),
            *_weight_specs(D, H2),
            *_weight_specs(D, H2),
        ],
        out_specs=[
            pl.BlockSpec((1, 8, R), cur3),
            pl.BlockSpec((1, P, T), prev3),
            pl.BlockSpec((1, 8, T * P), prev3),
            pl.BlockSpec((1, P, NB), cur3),
        ],
        out_shape=[
            jax.ShapeDtypeStruct((B, 8, R), jnp.float32),
            jax.ShapeDtypeStruct((B, P, T), jnp.int32),
            jax.ShapeDtypeStruct((B, 8, T * P), jnp.float32),
            jax.ShapeDtypeStruct((B, P, NB), jnp.float32),
        ],
        scratch_shapes=[pltpu.VMEM((P, NB), jnp.float32)],
        compiler_params=pltpu.CompilerParams(
            dimension_semantics=("parallel", "arbitrary")),
        cost_estimate=cost,
    )(phrase_embed, box_features, phrase_embed, box_features,
      main_w1pd, main_w1p, main_w1b, main_b1, main_w2t, main_b2,
      topn_w1pd, topn_w1p, topn_w1b, topn_b1, topn_w2t, topn_b2)

    out = out.reshape(B, 8, P, NB)
    reg_offset = jnp.transpose(out[:, 1:7], (0, 2, 3, 1))

    outt = jnp.transpose(outt.reshape(B, 8, T, P), (0, 1, 3, 2))
    pred_sim_topN = outt[:, 0]
    reg_offset_topN = jnp.transpose(outt[:, 1:7], (0, 2, 3, 1))

    bidx = jnp.arange(B)
    res = {
        "batch_pred_similarity":      pred_sim[bidx, target_ids],
        "batch_reg_offset":           reg_offset[bidx, target_ids],
        "batch_pred_similarity_topN": pred_sim_topN[bidx, target_ids],
        "batch_reg_offset_topN":      reg_offset_topN[bidx, target_ids],
        "batch_topN_target_ids":      ids[bidx, target_ids],
        "batch_target_embed":         phrase_embed[bidx, target_ids],
    }
    return res, (pred_sim, reg_offset, pred_sim_topN, reg_offset_topN, ids)
